# baseline (speedup 1.0000x reference)
import functools
import math

import jax
import jax.numpy as jnp
from jax.experimental import pallas as pl
from jax.experimental.pallas import tpu as pltpu

NEG = float(jnp.finfo(jnp.float32).min)
STREAMING_RATIO, SELECTING_RATIO, RECENT_RATIO = 0.02, 0.06, 0.12


def _k1_body(hs_ref, qw_ref, kw_ref, vw_ref, rot_ref, cos_ref, sin_ref,
             q_ref, k_ref, v_ref):
    x = hs_ref[...].astype(jnp.bfloat16)
    dnum_t = (((1,), (1,)), ((), ()))
    dnum_n = (((1,), (0,)), ((), ()))
    f32 = jnp.float32
    bf = jnp.bfloat16
    q = jax.lax.dot_general(x, qw_ref[...].astype(bf), dnum_t,
                            preferred_element_type=f32)
    k = jax.lax.dot_general(x, kw_ref[...].astype(bf), dnum_t,
                            preferred_element_type=f32)
    v = jax.lax.dot_general(x, vw_ref[...].astype(bf), dnum_t,
                            preferred_element_type=f32)
    rot = rot_ref[...]
    cos, sin = cos_ref[...], sin_ref[...]
    hp = jax.lax.Precision.HIGHEST
    qr = jax.lax.dot_general(q, rot, dnum_n, precision=hp,
                             preferred_element_type=f32)
    kr = jax.lax.dot_general(k, rot, dnum_n, precision=hp,
                             preferred_element_type=f32)
    q_ref[...] = q * cos + qr * sin
    k_ref[...] = k * cos + kr * sin
    v_ref[...] = v


def _k1(hs, q_w, k_w, v_w, rot, cos_f, sin_f, blk):
    S, D = hs.shape
    grid = (S // blk,)
    bs_x = pl.BlockSpec((blk, D), lambda i: (i, 0))
    bs_w = pl.BlockSpec((D, D), lambda i: (0, 0))
    out = jax.ShapeDtypeStruct((S, D), jnp.float32)
    return pl.pallas_call(
        _k1_body,
        grid=grid,
        in_specs=[bs_x, bs_w, bs_w, bs_w, bs_w, bs_x, bs_x],
        out_specs=[bs_x, bs_x, bs_x],
        out_shape=[out, out, out],
    )(hs, q_w, k_w, v_w, rot, cos_f, sin_f)


def _k2_body(q_ref, k_ref, probs_ref, ss0_ref, *, blk, cache, scale):
    r = pl.program_id(1)
    qb = q_ref[0].astype(jnp.bfloat16)
    kb = k_ref[0].astype(jnp.bfloat16)
    s = jax.lax.dot_general(qb, kb, (((1,), (1,)), ((), ())),
                            preferred_element_type=jnp.float32) * scale
    t = r * blk + jax.lax.broadcasted_iota(jnp.int32, (blk, 1), 0)
    j = jax.lax.broadcasted_iota(jnp.int32, s.shape, 1)
    s = jnp.where(j <= t, s, NEG)
    m = jnp.max(s, axis=1, keepdims=True)
    e = jnp.exp(s - m)
    z = jnp.sum(e, axis=1, keepdims=True)
    p = e / z
    probs_ref[0] = p
    contrib = jnp.sum(jnp.where(t >= cache, p, 0.0), axis=0, keepdims=True)

    @pl.when(r == 0)
    def _():
        ss0_ref[0] = contrib

    @pl.when(r > 0)
    def _():
        ss0_ref[0] += contrib


def _k2(q, k, blk, cache):
    H, S, DH = q.shape
    scale = 1.0 / math.sqrt(DH)
    grid = (H, S // blk)
    body = functools.partial(_k2_body, blk=blk, cache=cache, scale=scale)
    return pl.pallas_call(
        body,
        grid=grid,
        in_specs=[pl.BlockSpec((1, blk, DH), lambda h, r: (h, r, 0)),
                  pl.BlockSpec((1, S, DH), lambda h, r: (h, 0, 0))],
        out_specs=[pl.BlockSpec((1, blk, S), lambda h, r: (h, r, 0)),
                   pl.BlockSpec((1, 1, S), lambda h, r: (h, 0, 0))],
        out_shape=[jax.ShapeDtypeStruct((H, S, S), jnp.float32),
                   jax.ShapeDtypeStruct((H, 1, S), jnp.float32)],
    )(q, k)


def _k3_body(probs_ref, ss0_ref, v_ref, out_ref, ss_ref, pn_ref, *,
             blk, sb, rb, cache, S, H, DH):
    b = pl.program_id(0)
    INF = jnp.float32(jnp.inf)
    jb = jax.lax.broadcasted_iota(jnp.int32, (H, S), 1)
    jf = jb.astype(jnp.float32)

    @pl.when(b == 0)
    def _():
        ss_ref[...] = ss0_ref[:, 0, :]

    base = b * blk

    def step(i, _):
        t = base + i
        row = probs_ref[:, i, :]
        ss = ss_ref[...]
        alive = ss < INF
        rz = row / jnp.sum(jnp.where(alive, row, 0.0), axis=1, keepdims=True)
        pn = jnp.where(alive, rz, 0.0)
        pn_ref[:, i, :] = pn

        @pl.when((t >= cache) & (t <= S - 2))
        def _():
            ssn = ss + rz
            tf = t.astype(jnp.float32)
            tmp = ssn / ((tf + 1.0) - jf)
            valid = (jb >= sb) & (jb <= t - rb)
            tmp = jnp.where(valid, tmp, INF)
            m = jnp.min(tmp, axis=1, keepdims=True)
            cand = jnp.where(tmp == m, jb, S * 2)
            mi = jnp.min(cand, axis=1, keepdims=True)
            ss_ref[...] = jnp.where(jb == mi, INF, ssn)

        return 0

    jax.lax.fori_loop(0, blk, step, 0)

    for h in range(H):
        out_ref[h] = jax.lax.dot_general(
            pn_ref[h].astype(jnp.bfloat16), v_ref[h],
            (((1,), (0,)), ((), ())),
            preferred_element_type=jnp.float32)


def _k3(probs, ss0, v, blk, cache):
    H, S, _ = probs.shape
    DH = v.shape[2]
    sb = int(math.floor(STREAMING_RATIO * S + 0.5))
    rb = int(math.floor(RECENT_RATIO * S + 0.5))
    body = functools.partial(_k3_body, blk=blk, sb=sb, rb=rb,
                             cache=cache, S=S, H=H, DH=DH)
    return pl.pallas_call(
        body,
        grid=(S // blk,),
        in_specs=[pl.BlockSpec((H, blk, S), lambda b: (0, b, 0)),
                  pl.BlockSpec((H, 1, S), lambda b: (0, 0, 0)),
                  pl.BlockSpec((H, S, DH), lambda b: (0, 0, 0))],
        out_specs=pl.BlockSpec((H, blk, DH), lambda b: (0, b, 0)),
        out_shape=jax.ShapeDtypeStruct((H, S, DH), jnp.float32),
        scratch_shapes=[pltpu.VMEM((H, S), jnp.float32),
                        pltpu.VMEM((H, blk, S), jnp.float32)],
    )(probs, ss0, v)


def _k5_body(x_ref, w_ref, o_ref):
    o_ref[...] = jax.lax.dot_general(x_ref[...].astype(jnp.bfloat16),
                                     w_ref[...].astype(jnp.bfloat16),
                                     (((1,), (1,)), ((), ())),
                                     preferred_element_type=jnp.float32)


def _k5(x, w, blk):
    S, D = x.shape
    return pl.pallas_call(
        _k5_body,
        grid=(S // blk,),
        in_specs=[pl.BlockSpec((blk, D), lambda i: (i, 0)),
                  pl.BlockSpec((D, D), lambda i: (0, 0))],
        out_specs=pl.BlockSpec((blk, D), lambda i: (i, 0)),
        out_shape=jax.ShapeDtypeStruct((S, D), jnp.float32),
    )(x, w)


def kernel(hidden_states, attention_mask, position_ids, q_w, k_w, v_w, o_w):
    B, S, D = hidden_states.shape
    H = 16
    DH = D // H
    sb = int(math.floor(STREAMING_RATIO * S + 0.5))
    selb = int(math.floor(SELECTING_RATIO * S + 0.5))
    rb = int(math.floor(RECENT_RATIO * S + 0.5))
    cache = sb + selb + rb

    hs = hidden_states[0]

    inv_freq = 1.0 / (10000.0 ** (jnp.arange(0, DH, 2, dtype=jnp.float32) / DH))
    tpos = jnp.arange(S, dtype=jnp.float32)
    freqs = jnp.einsum('i,j->ij', tpos, inv_freq)
    emb = jnp.concatenate([freqs, freqs], axis=-1)
    cos = jnp.cos(emb)[position_ids[0]]
    sin = jnp.sin(emb)[position_ids[0]]
    cos_f = jnp.tile(cos, (1, H))
    sin_f = jnp.tile(sin, (1, H))

    half = DH // 2
    eye = jnp.eye(half, dtype=jnp.float32)
    zero = jnp.zeros((half, half), jnp.float32)
    r64 = jnp.block([[zero, eye], [-eye, zero]])
    rot = jnp.kron(jnp.eye(H, dtype=jnp.float32), r64)

    q2, k2, v2 = _k1(hs, q_w, k_w, v_w, rot, cos_f, sin_f, blk=256)

    def heads(x):
        return x.reshape(S, H, DH).transpose(1, 0, 2)

    qh, kh, vh = heads(q2), heads(k2), heads(v2)

    probs, ss0 = _k2(qh, kh, blk=128, cache=cache)
    oh = _k3(probs, ss0, vh.astype(jnp.bfloat16), blk=32, cache=cache)

    merged = oh.transpose(1, 0, 2).reshape(S, D)
    out = _k5(merged, o_w, blk=256)
    return out.reshape(B, S, D)


# speedup vs baseline: 1.0979x; 1.0979x over previous
import functools
import math

import jax
import jax.numpy as jnp
from jax.experimental import pallas as pl
from jax.experimental.pallas import tpu as pltpu

NEG = float(jnp.finfo(jnp.float32).min)
STREAMING_RATIO, SELECTING_RATIO, RECENT_RATIO = 0.02, 0.06, 0.12


def _k1_body(hs_ref, qw_ref, kw_ref, vw_ref, cos_ref, sin_ref,
             q_ref, k_ref, v_ref, *, D):
    x = hs_ref[...].astype(jnp.bfloat16)
    dnum_t = (((1,), (1,)), ((), ()))
    f32 = jnp.float32
    bf = jnp.bfloat16
    q = jax.lax.dot_general(x, qw_ref[...].astype(bf), dnum_t,
                            preferred_element_type=f32)
    k = jax.lax.dot_general(x, kw_ref[...].astype(bf), dnum_t,
                            preferred_element_type=f32)
    v = jax.lax.dot_general(x, vw_ref[...].astype(bf), dnum_t,
                            preferred_element_type=f32)
    cos, sin = cos_ref[...], sin_ref[...]
    half = D // 2
    qr = jnp.concatenate([-q[:, half:], q[:, :half]], axis=1)
    kr = jnp.concatenate([-k[:, half:], k[:, :half]], axis=1)
    q_ref[...] = q * cos + qr * sin
    k_ref[...] = k * cos + kr * sin
    v_ref[...] = v


def _k1(hs, q_wp, k_wp, v_w, cos_f, sin_f, blk):
    S, D = hs.shape
    grid = (S // blk,)
    bs_x = pl.BlockSpec((blk, D), lambda i: (i, 0))
    bs_w = pl.BlockSpec((D, D), lambda i: (0, 0))
    out = jax.ShapeDtypeStruct((S, D), jnp.float32)
    return pl.pallas_call(
        functools.partial(_k1_body, D=D),
        grid=grid,
        in_specs=[bs_x, bs_w, bs_w, bs_w, bs_x, bs_x],
        out_specs=[bs_x, bs_x, bs_x],
        out_shape=[out, out, out],
    )(hs, q_wp, k_wp, v_w, cos_f, sin_f)


def _k2_body(q_ref, k_ref, probs_ref, ss0_ref, *, blk, cache, scale):
    r = pl.program_id(1)
    qb = q_ref[0].astype(jnp.bfloat16)
    kb = k_ref[0].astype(jnp.bfloat16)
    s = jax.lax.dot_general(qb, kb, (((1,), (1,)), ((), ())),
                            preferred_element_type=jnp.float32) * scale
    t = r * blk + jax.lax.broadcasted_iota(jnp.int32, (blk, 1), 0)
    j = jax.lax.broadcasted_iota(jnp.int32, s.shape, 1)
    s = jnp.where(j <= t, s, NEG)
    m = jnp.max(s, axis=1, keepdims=True)
    e = jnp.exp(s - m)
    z = jnp.sum(e, axis=1, keepdims=True)
    p = e / z
    probs_ref[0] = p
    contrib = jnp.sum(jnp.where(t >= cache, p, 0.0), axis=0, keepdims=True)

    @pl.when(r == 0)
    def _():
        ss0_ref[0] = contrib

    @pl.when(r > 0)
    def _():
        ss0_ref[0] += contrib


def _k2(q, k, blk, cache):
    H, S, DH = q.shape
    scale = 1.0 / math.sqrt(DH)
    grid = (H, S // blk)
    body = functools.partial(_k2_body, blk=blk, cache=cache, scale=scale)
    return pl.pallas_call(
        body,
        grid=grid,
        in_specs=[pl.BlockSpec((1, blk, DH), lambda h, r: (h, r, 0)),
                  pl.BlockSpec((1, S, DH), lambda h, r: (h, 0, 0))],
        out_specs=[pl.BlockSpec((1, blk, S), lambda h, r: (h, r, 0)),
                   pl.BlockSpec((1, 1, S), lambda h, r: (h, 0, 0))],
        out_shape=[jax.ShapeDtypeStruct((H, S, S), jnp.float32),
                   jax.ShapeDtypeStruct((H, 1, S), jnp.float32)],
    )(q, k)


def _k3_body(probs_ref, ss0_ref, et_ref, ss_ref, *, blk, row0, sb, rb,
             cache, S, H, nblk):
    b = pl.program_id(0)
    INF = jnp.float32(jnp.inf)
    jb = jax.lax.broadcasted_iota(jnp.int32, (H, S), 1)
    jf = jb.astype(jnp.float32)

    @pl.when(b == 0)
    def _():
        ss_ref[...] = ss0_ref[:, 0, :]
        et_ref[...] = jnp.full((H, S), S, dtype=jnp.int32)

    base = row0 + b * blk

    def step(i, _):
        t = base + i
        row = probs_ref[:, i, :]
        ss = ss_ref[...]
        zc = jnp.where(ss < INF, row, 0.0)
        z = jnp.sum(zc, axis=1, keepdims=True)
        ss = ss + row / z
        tf = t.astype(jnp.float32)
        tmp = ss / ((tf + 1.0) - jf)
        valid = (jb >= sb) & (jb <= t - rb)
        tmp = jnp.where(valid, tmp, INF)
        m = jnp.min(tmp, axis=1, keepdims=True)
        cand = jnp.where(tmp == m, jb, S * 2)
        mi = jnp.min(cand, axis=1, keepdims=True)
        ss_ref[...] = jnp.where(jb == mi, INF, ss)
        et_ref[...] = jnp.where(jb == mi, t, et_ref[...])
        return 0

    lo = jnp.maximum(0, cache - base)
    hi = jnp.minimum(blk, (S - 2) - base + 1)
    jax.lax.fori_loop(lo, hi, step, 0)


def _k3(probs, ss0, blk, cache):
    H, S, _ = probs.shape
    sb = int(math.floor(STREAMING_RATIO * S + 0.5))
    rb = int(math.floor(RECENT_RATIO * S + 0.5))
    row0 = (cache // blk) * blk
    nblk = (S - row0) // blk
    body = functools.partial(_k3_body, blk=blk, row0=row0, sb=sb, rb=rb,
                             cache=cache, S=S, H=H, nblk=nblk)
    return pl.pallas_call(
        body,
        grid=(nblk,),
        in_specs=[pl.BlockSpec((H, blk, S), lambda b: (0, b + row0 // blk, 0)),
                  pl.BlockSpec((H, 1, S), lambda b: (0, 0, 0))],
        out_specs=pl.BlockSpec((H, S), lambda b: (0, 0)),
        out_shape=jax.ShapeDtypeStruct((H, S), jnp.int32),
        scratch_shapes=[pltpu.VMEM((H, S), jnp.float32)],
    )(probs, ss0)


def _k4_body(probs_ref, et_ref, v_ref, out_ref, *, blk):
    qb = pl.program_id(1)
    p = probs_ref[0]
    et = et_ref[0]
    t = qb * blk + jax.lax.broadcasted_iota(jnp.int32, (blk, 1), 0)
    keep = et >= t
    pm = jnp.where(keep, p, 0.0)
    zs = jnp.sum(pm, axis=1, keepdims=True)
    pn = (pm / zs).astype(jnp.bfloat16)
    o = jax.lax.dot_general(pn, v_ref[0].astype(jnp.bfloat16),
                            (((1,), (0,)), ((), ())),
                            preferred_element_type=jnp.float32)
    out_ref[0] = o


def _k4(probs, et, v, blk):
    H, S, _ = probs.shape
    DH = v.shape[2]
    body = functools.partial(_k4_body, blk=blk)
    return pl.pallas_call(
        body,
        grid=(H, S // blk),
        in_specs=[pl.BlockSpec((1, blk, S), lambda h, q: (h, q, 0)),
                  pl.BlockSpec((1, 1, S), lambda h, q: (h, 0, 0)),
                  pl.BlockSpec((1, S, DH), lambda h, q: (h, 0, 0))],
        out_specs=pl.BlockSpec((1, blk, DH), lambda h, q: (h, q, 0)),
        out_shape=jax.ShapeDtypeStruct((H, S, DH), jnp.float32),
    )(probs, et, v)


def _k5_body(x_ref, w_ref, o_ref):
    o_ref[...] = jax.lax.dot_general(x_ref[...].astype(jnp.bfloat16),
                                     w_ref[...].astype(jnp.bfloat16),
                                     (((1,), (1,)), ((), ())),
                                     preferred_element_type=jnp.float32)


def _k5(x, w, blk):
    S, D = x.shape
    return pl.pallas_call(
        _k5_body,
        grid=(S // blk,),
        in_specs=[pl.BlockSpec((blk, D), lambda i: (i, 0)),
                  pl.BlockSpec((D, D), lambda i: (0, 0))],
        out_specs=pl.BlockSpec((blk, D), lambda i: (i, 0)),
        out_shape=jax.ShapeDtypeStruct((S, D), jnp.float32),
    )(x, w)


def kernel(hidden_states, attention_mask, position_ids, q_w, k_w, v_w, o_w):
    B, S, D = hidden_states.shape
    H = 16
    DH = D // H
    sb = int(math.floor(STREAMING_RATIO * S + 0.5))
    selb = int(math.floor(SELECTING_RATIO * S + 0.5))
    rb = int(math.floor(RECENT_RATIO * S + 0.5))
    cache = sb + selb + rb

    hs = hidden_states[0]

    inv_freq = 1.0 / (10000.0 ** (jnp.arange(0, DH, 2, dtype=jnp.float32) / DH))
    tpos = jnp.arange(S, dtype=jnp.float32)
    freqs = jnp.einsum('i,j->ij', tpos, inv_freq)
    emb = jnp.concatenate([freqs, freqs], axis=-1)
    cos = jnp.cos(emb)[position_ids[0]]
    sin = jnp.sin(emb)[position_ids[0]]
    half = DH // 2
    cos_f = jnp.tile(cos[:, :half], (1, D // half))
    sin_f = jnp.tile(sin[:, :half], (1, D // half))
    q_wp = q_w.reshape(H, 2, half, D).transpose(1, 0, 2, 3).reshape(D, D)
    k_wp = k_w.reshape(H, 2, half, D).transpose(1, 0, 2, 3).reshape(D, D)

    q2, k2, v2 = _k1(hs, q_wp, k_wp, v_w, cos_f, sin_f, blk=256)

    def heads_perm(x):
        return x.reshape(S, 2, H, half).transpose(2, 0, 1, 3).reshape(H, S, DH)

    qh = heads_perm(q2)
    kh = heads_perm(k2)
    vh = v2.reshape(S, H, DH).transpose(1, 0, 2)

    probs, ss0 = _k2(qh, kh, blk=128, cache=cache)
    et = _k3(probs, ss0, blk=32, cache=cache)
    oh = _k4(probs, et.reshape(H, 1, S), vh, blk=256)

    merged = oh.transpose(1, 0, 2).reshape(S, D)
    out = _k5(merged, o_w, blk=256)
    return out.reshape(B, S, D)
